# baseline (speedup 1.0000x reference)
import jax
import jax.numpy as jnp
from jax import lax
from jax.experimental import pallas as pl
from jax.experimental.pallas import tpu as pltpu
from jax.experimental.pallas import tpu_sc as plsc

N_FIELDS = 22
VOCAB = 12
EMB_DIM = 64
BATCH = 16384

TOTAL_ROWS = BATCH * N_FIELDS
NUM_WORKERS = 32
ROWS_PER_WORKER = TOTAL_ROWS // NUM_WORKERS
CHUNK = 256
NCHUNKS = ROWS_PER_WORKER // CHUNK
LANES = 16
GROUPS = CHUNK // LANES
NBUF = 4
NITER = NCHUNKS // NBUF
TABLE_WORDS = N_FIELDS * VOCAB * EMB_DIM


def _body(xflat_hbm, table_hbm, out_hbm, tablebuf, idxbuf, outbuf, *sems):
    wsem = sems
    wid = lax.axis_index("s") * 2 + lax.axis_index("c")
    wbase = wid * ROWS_PER_WORKER
    lane = lax.iota(jnp.int32, LANES)

    pltpu.sync_copy(table_hbm, tablebuf)
    pltpu.sync_copy(xflat_hbm.at[pl.ds(wbase, ROWS_PER_WORKER)], idxbuf)

    offs = [
        (((j * LANES + lane) % N_FIELDS) * (VOCAB * EMB_DIM))
        for j in range(11)
    ]

    def id_body(r, c):
        base = r * (11 * LANES)
        for j in range(11):
            s = base + j * LANES
            idxbuf[pl.ds(s, LANES)] = idxbuf[pl.ds(s, LANES)] * EMB_DIM + offs[j]
        return c

    lax.fori_loop(0, ROWS_PER_WORKER // (11 * LANES), id_body, 0)

    lane64 = lane * EMB_DIM
    one = jnp.full((LANES,), 1, jnp.int32)

    def compute_chunk(slot, k):
        def group_body(g, c):
            src = idxbuf[pl.ds(k * CHUNK + g * LANES, LANES)]
            dst = g * (LANES * EMB_DIM) + lane64
            for d in range(EMB_DIM):
                v = plsc.load_gather(tablebuf, [src])
                plsc.store_scatter(outbuf.at[slot], [dst], v)
                if d != EMB_DIM - 1:
                    src = src + one
                    dst = dst + one
            return c

        lax.fori_loop(0, GROUPS, group_body, 0)

    def w_start(slot, k):
        pltpu.async_copy(
            outbuf.at[slot],
            out_hbm.at[pl.ds((wbase + k * CHUNK) * EMB_DIM, CHUNK * EMB_DIM)],
            wsem[slot])

    def w_wait(slot, k):
        pltpu.make_async_copy(
            outbuf.at[slot],
            out_hbm.at[pl.ds((wbase + k * CHUNK) * EMB_DIM, CHUNK * EMB_DIM)],
            wsem[slot]).wait()

    def block(k0, c):
        for b in range(NBUF):
            @pl.when(k0 > 0)
            def _():
                w_wait(b, (k0 - 1) * NBUF + b)

            compute_chunk(b, k0 * NBUF + b)
            w_start(b, k0 * NBUF + b)
        return c

    lax.fori_loop(0, NITER, block, 0)

    for b in range(NBUF):
        w_wait(b, (NITER - 1) * NBUF + b)


@jax.jit
def _gather(xflat, table):
    mesh = plsc.VectorSubcoreMesh(core_axis_name="c", subcore_axis_name="s")
    return pl.kernel(
        _body,
        out_type=jax.ShapeDtypeStruct((TOTAL_ROWS * EMB_DIM,), jnp.float32),
        mesh=mesh,
        scratch_types=[
            pltpu.VMEM((TABLE_WORDS,), jnp.float32),
            pltpu.VMEM((ROWS_PER_WORKER,), jnp.int32),
            pltpu.VMEM((NBUF, CHUNK * EMB_DIM), jnp.float32),
        ] + [pltpu.SemaphoreType.DMA] * NBUF,
        compiler_params=pltpu.CompilerParams(
            use_tc_tiling_on_sc=False, needs_layout_passes=False),
    )(xflat, table)


def kernel(x, W):
    xflat = x.reshape(-1).astype(jnp.int32)
    table = W.reshape(-1)
    out = _gather(xflat, table)
    return out.reshape(BATCH, N_FIELDS * EMB_DIM)


# speedup vs baseline: 4.3636x; 4.3636x over previous
import jax
import jax.numpy as jnp
from jax import lax
from jax.experimental import pallas as pl
from jax.experimental.pallas import tpu as pltpu
from jax.experimental.pallas import tpu_sc as plsc

N_FIELDS = 22
VOCAB = 12
EMB_DIM = 64
BATCH = 16384

N_PAIRS = N_FIELDS // 2
PAIR_DIM = 2 * EMB_DIM
PAIR_VOCAB = VOCAB * VOCAB
TOTAL_PROWS = BATCH * N_PAIRS
NUM_WORKERS = 32
PROWS_PER_WORKER = TOTAL_PROWS // NUM_WORKERS
CHUNK = 128
NCHUNKS = PROWS_PER_WORKER // CHUNK
LANES = 16
NBUF = 4
NITER = NCHUNKS // NBUF


def _body(xflat_hbm, ptable_hbm, out_hbm, xbuf, idxbuf, rowsbuf, *sems):
    gsem = sems[:NBUF]
    wsem = sems[NBUF:]
    wid = lax.axis_index("s") * 2 + lax.axis_index("c")
    pbase = wid * PROWS_PER_WORKER
    lane = lax.iota(jnp.int32, LANES)

    pltpu.sync_copy(
        xflat_hbm.at[pl.ds(pbase * 2, PROWS_PER_WORKER * 2)], xbuf)

    offs = [
        (((j * LANES + lane) % N_PAIRS) * PAIR_VOCAB)
        for j in range(11)
    ]
    lane2 = lane * 2

    def id_body(r, c):
        base = r * (11 * LANES)
        for j in range(11):
            p = base + j * LANES
            ev = plsc.load_gather(xbuf, [jnp.full((LANES,), 2 * p, jnp.int32) + lane2])
            od = plsc.load_gather(xbuf, [jnp.full((LANES,), 2 * p + 1, jnp.int32) + lane2])
            idxbuf[pl.ds(p, LANES)] = ev * VOCAB + od + offs[j]
        return c

    lax.fori_loop(0, PROWS_PER_WORKER // (11 * LANES), id_body, 0)

    def g_start(slot, g):
        pltpu.async_copy(
            ptable_hbm.at[idxbuf.at[pl.ds(g * CHUNK, CHUNK)]],
            rowsbuf.at[slot], gsem[slot])

    def g_wait(slot, g):
        pltpu.make_async_copy(
            ptable_hbm.at[idxbuf.at[pl.ds(g * CHUNK, CHUNK)]],
            rowsbuf.at[slot], gsem[slot]).wait()

    def w_start(slot, g):
        pltpu.async_copy(
            rowsbuf.at[slot],
            out_hbm.at[pl.ds(pbase + g * CHUNK, CHUNK)], wsem[slot])

    def w_wait(slot, g):
        pltpu.make_async_copy(
            rowsbuf.at[slot],
            out_hbm.at[pl.ds(pbase + g * CHUNK, CHUNK)], wsem[slot]).wait()

    for b in range(NBUF):
        g_start(b, b)

    def block(k, c):
        for b in range(NBUF):
            g_wait(b, k * NBUF + b)
            w_start(b, k * NBUF + b)

        @pl.when(k < NITER - 1)
        def _():
            for b in range(NBUF):
                w_wait(b, k * NBUF + b)
                g_start(b, (k + 1) * NBUF + b)

        return c

    lax.fori_loop(0, NITER, block, 0)

    for b in range(NBUF):
        w_wait(b, (NITER - 1) * NBUF + b)


@jax.jit
def _gather(xflat, ptable):
    mesh = plsc.VectorSubcoreMesh(core_axis_name="c", subcore_axis_name="s")
    return pl.kernel(
        _body,
        out_type=jax.ShapeDtypeStruct((TOTAL_PROWS, PAIR_DIM), jnp.float32),
        mesh=mesh,
        scratch_types=[
            pltpu.VMEM((PROWS_PER_WORKER * 2,), jnp.int32),
            pltpu.VMEM((PROWS_PER_WORKER,), jnp.int32),
            pltpu.VMEM((NBUF, CHUNK, PAIR_DIM), jnp.float32),
        ] + [pltpu.SemaphoreType.DMA] * (2 * NBUF),
        compiler_params=pltpu.CompilerParams(
            use_tc_tiling_on_sc=False, needs_layout_passes=False),
    )(xflat, ptable)


def kernel(x, W):
    xflat = x.reshape(-1).astype(jnp.int32)
    We = jnp.broadcast_to(W[0::2][:, :, None, :], (N_PAIRS, VOCAB, VOCAB, EMB_DIM))
    Wo = jnp.broadcast_to(W[1::2][:, None, :, :], (N_PAIRS, VOCAB, VOCAB, EMB_DIM))
    ptable = jnp.concatenate([We, Wo], axis=-1).reshape(N_PAIRS * PAIR_VOCAB, PAIR_DIM)
    out = _gather(xflat, ptable)
    return out.reshape(BATCH, N_FIELDS * EMB_DIM)


# speedup vs baseline: 7.4596x; 1.7095x over previous
import jax
import jax.numpy as jnp
from jax import lax
from jax.experimental import pallas as pl
from jax.experimental.pallas import tpu as pltpu
from jax.experimental.pallas import tpu_sc as plsc

N_FIELDS = 22
VOCAB = 12
EMB_DIM = 64
BATCH = 16384

N_PAIRS = N_FIELDS // 2
PAIR_DIM = 2 * EMB_DIM
PAIR_VOCAB = VOCAB * VOCAB
TOTAL_PROWS = BATCH * N_PAIRS
NUM_WORKERS = 32
PROWS_PER_WORKER = TOTAL_PROWS // NUM_WORKERS
ROWS_PER_WORKER = BATCH // NUM_WORKERS
STRIPE = 88
NCHUNKS = PROWS_PER_WORKER // STRIPE
LANES = 16
NBUF = 4
NITER = NCHUNKS // NBUF


def _body(xflat_hbm, ptable_hbm, out_hbm, xbuf, idxbuf, rowsbuf, *sems):
    gsem = sems[:NBUF]
    wsem = sems[NBUF:]
    wid = lax.axis_index("s") * 2 + lax.axis_index("c")
    row0 = wid * ROWS_PER_WORKER
    lane = lax.iota(jnp.int32, LANES)

    pltpu.sync_copy(
        xflat_hbm.at[pl.ds(row0 * N_FIELDS, ROWS_PER_WORKER * N_FIELDS)],
        xbuf)

    EV = []
    TV = []
    for j in range(11):
        q = j * LANES + lane
        m = q % STRIPE
        t = m // 8
        i = m % 8
        cc = q // STRIPE
        EV.append((8 * cc + i) * N_FIELDS + 2 * t)
        TV.append(t * PAIR_VOCAB)

    def id_body(r, c):
        base = jnp.full((LANES,), 352 * r, jnp.int32)
        for j in range(11):
            ev = plsc.load_gather(xbuf, [base + EV[j]])
            od = plsc.load_gather(xbuf, [base + (EV[j] + 1)])
            idxbuf[pl.ds(r * 176 + j * LANES, LANES)] = ev * VOCAB + od + TV[j]
        return c

    lax.fori_loop(0, PROWS_PER_WORKER // 176, id_body, 0)

    def g_start(slot, g):
        pltpu.async_copy(
            ptable_hbm.at[idxbuf.at[pl.ds(g * STRIPE, STRIPE)]],
            rowsbuf.at[slot], gsem[slot])

    def g_wait(slot, g):
        pltpu.make_async_copy(
            ptable_hbm.at[idxbuf.at[pl.ds(g * STRIPE, STRIPE)]],
            rowsbuf.at[slot], gsem[slot]).wait()

    pbase = wid * PROWS_PER_WORKER

    def w_start(slot, g):
        pltpu.async_copy(
            rowsbuf.at[slot],
            out_hbm.at[pl.ds(pbase + g * STRIPE, STRIPE)], wsem[slot])

    def w_wait(slot, g):
        pltpu.make_async_copy(
            rowsbuf.at[slot],
            out_hbm.at[pl.ds(pbase + g * STRIPE, STRIPE)], wsem[slot]).wait()

    for b in range(NBUF):
        g_start(b, b)

    def block(k, c):
        for b in range(NBUF):
            g_wait(b, k * NBUF + b)
            w_start(b, k * NBUF + b)

        @pl.when(k < NITER - 1)
        def _():
            for b in range(NBUF):
                w_wait(b, k * NBUF + b)
                g_start(b, (k + 1) * NBUF + b)

        return c

    lax.fori_loop(0, NITER, block, 0)

    for b in range(NBUF):
        w_wait(b, (NITER - 1) * NBUF + b)


@jax.jit
def _gather(xflat, ptable):
    mesh = plsc.VectorSubcoreMesh(core_axis_name="c", subcore_axis_name="s")
    return pl.kernel(
        _body,
        out_type=jax.ShapeDtypeStruct((TOTAL_PROWS, PAIR_DIM), jnp.float32),
        mesh=mesh,
        scratch_types=[
            pltpu.VMEM((ROWS_PER_WORKER * N_FIELDS,), jnp.int32),
            pltpu.VMEM((PROWS_PER_WORKER,), jnp.int32),
            pltpu.VMEM((NBUF, STRIPE, PAIR_DIM), jnp.float32),
        ] + [pltpu.SemaphoreType.DMA] * (2 * NBUF),
        compiler_params=pltpu.CompilerParams(
            use_tc_tiling_on_sc=False, needs_layout_passes=False),
    )(xflat, ptable)


def kernel(x, W):
    xflat = x.reshape(-1).astype(jnp.int32)
    We = jnp.broadcast_to(W[0::2][:, :, None, :], (N_PAIRS, VOCAB, VOCAB, EMB_DIM))
    Wo = jnp.broadcast_to(W[1::2][:, None, :, :], (N_PAIRS, VOCAB, VOCAB, EMB_DIM))
    ptable = jnp.concatenate([We, Wo], axis=-1).reshape(N_PAIRS * PAIR_VOCAB, PAIR_DIM)
    out = _gather(xflat, ptable)
    return (out.reshape(BATCH // 8, N_PAIRS, 8, PAIR_DIM)
            .transpose(0, 2, 1, 3)
            .reshape(BATCH, N_FIELDS * EMB_DIM))


# speedup vs baseline: 7.5261x; 1.0089x over previous
import jax
import jax.numpy as jnp
from jax import lax
from jax.experimental import pallas as pl
from jax.experimental.pallas import tpu as pltpu
from jax.experimental.pallas import tpu_sc as plsc

N_FIELDS = 22
VOCAB = 12
EMB_DIM = 64
BATCH = 16384

N_PAIRS = N_FIELDS // 2
PAIR_DIM = 2 * EMB_DIM
PAIR_VOCAB = VOCAB * VOCAB
TOTAL_PROWS = BATCH * N_PAIRS
NUM_WORKERS = 32
PROWS_PER_WORKER = TOTAL_PROWS // NUM_WORKERS
ROWS_PER_WORKER = BATCH // NUM_WORKERS
STRIPE = 88
NCHUNKS = PROWS_PER_WORKER // STRIPE
LANES = 16
NBUF = 8
NITER = NCHUNKS // NBUF


def _body(xflat_hbm, ptable_hbm, out_hbm, xbuf, idxbuf, rowsbuf, *sems):
    gsem = sems[:NBUF]
    wsem = sems[NBUF:]
    wid = lax.axis_index("s") * 2 + lax.axis_index("c")
    row0 = wid * ROWS_PER_WORKER
    lane = lax.iota(jnp.int32, LANES)

    pltpu.sync_copy(
        xflat_hbm.at[pl.ds(row0 * N_FIELDS, ROWS_PER_WORKER * N_FIELDS)],
        xbuf)

    EV = []
    TV = []
    for j in range(11):
        q = j * LANES + lane
        m = q % STRIPE
        t = m // 8
        i = m % 8
        cc = q // STRIPE
        EV.append((8 * cc + i) * N_FIELDS + 2 * t)
        TV.append(t * PAIR_VOCAB)

    def id_body(r, c):
        base = jnp.full((LANES,), 352 * r, jnp.int32)
        for j in range(11):
            ev = plsc.load_gather(xbuf, [base + EV[j]])
            od = plsc.load_gather(xbuf, [base + (EV[j] + 1)])
            idxbuf[pl.ds(r * 176 + j * LANES, LANES)] = ev * VOCAB + od + TV[j]
        return c

    lax.fori_loop(0, PROWS_PER_WORKER // 176, id_body, 0)

    def g_start(slot, g):
        pltpu.async_copy(
            ptable_hbm.at[idxbuf.at[pl.ds(g * STRIPE, STRIPE)]],
            rowsbuf.at[slot], gsem[slot])

    def g_wait(slot, g):
        pltpu.make_async_copy(
            ptable_hbm.at[idxbuf.at[pl.ds(g * STRIPE, STRIPE)]],
            rowsbuf.at[slot], gsem[slot]).wait()

    pbase = wid * PROWS_PER_WORKER

    def w_start(slot, g):
        pltpu.async_copy(
            rowsbuf.at[slot],
            out_hbm.at[pl.ds(pbase + g * STRIPE, STRIPE)], wsem[slot])

    def w_wait(slot, g):
        pltpu.make_async_copy(
            rowsbuf.at[slot],
            out_hbm.at[pl.ds(pbase + g * STRIPE, STRIPE)], wsem[slot]).wait()

    for b in range(NBUF):
        g_start(b, b)

    def block(k, c):
        for b in range(NBUF):
            g_wait(b, k * NBUF + b)
            w_start(b, k * NBUF + b)

        @pl.when(k < NITER - 1)
        def _():
            for b in range(NBUF):
                w_wait(b, k * NBUF + b)
                g_start(b, (k + 1) * NBUF + b)

        return c

    lax.fori_loop(0, NITER, block, 0)

    for b in range(NBUF):
        w_wait(b, (NITER - 1) * NBUF + b)


@jax.jit
def _gather(xflat, ptable):
    mesh = plsc.VectorSubcoreMesh(core_axis_name="c", subcore_axis_name="s")
    return pl.kernel(
        _body,
        out_type=jax.ShapeDtypeStruct((TOTAL_PROWS, PAIR_DIM), jnp.float32),
        mesh=mesh,
        scratch_types=[
            pltpu.VMEM((ROWS_PER_WORKER * N_FIELDS,), jnp.int32),
            pltpu.VMEM((PROWS_PER_WORKER,), jnp.int32),
            pltpu.VMEM((NBUF, STRIPE, PAIR_DIM), jnp.float32),
        ] + [pltpu.SemaphoreType.DMA] * (2 * NBUF),
        compiler_params=pltpu.CompilerParams(
            use_tc_tiling_on_sc=False, needs_layout_passes=False),
    )(xflat, ptable)


def kernel(x, W):
    xflat = x.reshape(-1).astype(jnp.int32)
    We = jnp.broadcast_to(W[0::2][:, :, None, :], (N_PAIRS, VOCAB, VOCAB, EMB_DIM))
    Wo = jnp.broadcast_to(W[1::2][:, None, :, :], (N_PAIRS, VOCAB, VOCAB, EMB_DIM))
    ptable = jnp.concatenate([We, Wo], axis=-1).reshape(N_PAIRS * PAIR_VOCAB, PAIR_DIM)
    out = _gather(xflat, ptable)
    return (out.reshape(BATCH // 8, N_PAIRS, 8, PAIR_DIM)
            .transpose(0, 2, 1, 3)
            .reshape(BATCH, N_FIELDS * EMB_DIM))
